# baseline (speedup 1.0000x reference)
import jax
import jax.numpy as jnp
from jax import lax
from jax.experimental import pallas as pl
from jax.experimental.pallas import tpu as pltpu
from jax.experimental.pallas import tpu_sc as plsc

_NC, _NS = 2, 16
_NW = _NC * _NS
_B, _T = 16384, 200
_BW = _B // _NW
_RT = 8
_NSI = _T // _RT
_OT = 40
_NSO = _T // _OT
_DI = 8
_G = _RT * (_BW // 16)


def _psk_body(zt_hbm, ct_hbm, out_hbm, tabc_v, tabs_v,
              z0, z1, z2, z3, z4, z5, z6, z7, o0, o1,
              si0, si1, si2, si3, si4, si5, si6, si7, so0, so1, tsem):
    wid = lax.axis_index("s") * _NC + lax.axis_index("c")
    b0 = wid * _BW

    zbuf = (z0, z1, z2, z3, z4, z5, z6, z7)
    zsem = (si0, si1, si2, si3, si4, si5, si6, si7)
    obuf, osem = (o0, o1), (so0, so1)

    def in_copy(si, p):
        return pltpu.make_async_copy(
            zt_hbm.at[pl.ds(si * _RT, _RT), pl.ds(b0, _BW)], zbuf[p], zsem[p])

    def out_copy(so, q):
        return pltpu.make_async_copy(
            obuf[q], out_hbm.at[pl.ds(so * _OT, _OT), pl.ds(8 * wid, 8), :],
            osem[q])

    def compute(p, q, sub):
        zv_ref, ov_ref = zbuf[p], obuf[q]

        @plsc.parallel_loop(0, _G, unroll=8)
        def _grp(i):
            t2 = sub * _RT + (i >> 5)
            g = i & 31
            zv = zv_ref[i >> 5, pl.ds(g * 16, 16)]
            cv = plsc.load_gather(tabc_v, [zv])
            sv = plsc.load_gather(tabs_v, [zv])
            bt = g >> 3
            j = g & 7
            ov_ref[t2, 2 * bt, pl.ds(j * 16, 16)] = cv
            ov_ref[t2, 2 * bt + 1, pl.ds(j * 16, 16)] = sv

    tabc_cp = pltpu.make_async_copy(ct_hbm.at[0], tabc_v, tsem)
    tabs_cp = pltpu.make_async_copy(ct_hbm.at[1], tabs_v, tsem)
    tabc_cp.start()
    tabs_cp.start()
    for p in range(_DI):
        in_copy(p, p).start()
    tabc_cp.wait()
    tabs_cp.wait()

    for so in range(_NSO):
        q = so % 2
        if so >= 2:
            out_copy(so - 2, q).wait()
        for sub in range(_NSO):
            si = so * _NSO + sub
            p = si % _DI
            in_copy(si, p).wait()
            compute(p, q, sub)
            if si + _DI <= _NSI - 1:
                in_copy(si + _DI, p).start()
        out_copy(so, q).start()
    out_copy(_NSO - 2, (_NSO - 2) % 2).wait()
    out_copy(_NSO - 1, (_NSO - 1) % 2).wait()


def kernel(z, constellation):
    zt = z.T
    ct = constellation.T
    out3 = pl.kernel(
        _psk_body,
        out_type=jax.ShapeDtypeStruct((_T, 2 * _B // 128, 128), jnp.float32),
        mesh=plsc.VectorSubcoreMesh(
            core_axis_name="c", subcore_axis_name="s",
            num_cores=_NC, num_subcores=_NS,
        ),
        scratch_types=(
            [pltpu.VMEM((16,), jnp.float32)] * 2
            + [pltpu.VMEM((_RT, _BW), jnp.int32)] * _DI
            + [pltpu.VMEM((_OT, 8, 128), jnp.float32)] * 2
            + [pltpu.SemaphoreType.DMA] * (_DI + 3)
        ),
        compiler_params=pltpu.CompilerParams(
            needs_layout_passes=False, use_tc_tiling_on_sc=True,
        ),
    )(zt, ct)
    out = out3.reshape(_T, 128, 2, 128).transpose(1, 3, 0, 2).reshape(_B, _T, 2)
    return out


# speedup vs baseline: 1.0533x; 1.0533x over previous
import jax
import jax.numpy as jnp
from jax import lax
from jax.experimental import pallas as pl
from jax.experimental.pallas import tpu as pltpu
from jax.experimental.pallas import tpu_sc as plsc

_NC, _NS = 2, 16
_NW = _NC * _NS
_B, _T = 16384, 200
_BW = _B // _NW
_RT = 40
_NST = _T // _RT
_G = _RT * (_BW // 16)


def _psk_body(zt_hbm, ct_hbm, out_hbm, tabc_v, tabs_v,
              z0, z1, o0, o1, si0, si1, so0, so1):
    wid = lax.axis_index("s") * _NC + lax.axis_index("c")
    b0 = wid * _BW

    zbuf, obuf = (z0, z1), (o0, o1)
    zsem, osem = (si0, si1), (so0, so1)

    def in_copy(si, p):
        return pltpu.make_async_copy(
            zt_hbm.at[pl.ds(si * _RT, _RT), pl.ds(b0, _BW)], zbuf[p], zsem[p])

    def out_copy(si, p):
        return pltpu.make_async_copy(
            obuf[p], out_hbm.at[pl.ds(si * _RT, _RT), pl.ds(8 * wid, 8), :],
            osem[p])

    def compute(p):
        zv_ref, ov_ref = zbuf[p], obuf[p]

        @plsc.parallel_loop(0, _G, unroll=8)
        def _grp(i):
            t2 = i >> 5
            g = i & 31
            zv = zv_ref[t2, pl.ds(g * 16, 16)]
            cv = plsc.load_gather(tabc_v, [zv])
            sv = plsc.load_gather(tabs_v, [zv])
            bt = g >> 3
            j = g & 7
            ov_ref[t2, 2 * bt, pl.ds(j * 16, 16)] = cv
            ov_ref[t2, 2 * bt + 1, pl.ds(j * 16, 16)] = sv

    tabc_cp = pltpu.make_async_copy(ct_hbm.at[0], tabc_v, si0)
    tabs_cp = pltpu.make_async_copy(ct_hbm.at[1], tabs_v, si1)
    tabc_cp.start()
    tabs_cp.start()
    in_copy(0, 0).start()
    in_copy(1, 1).start()
    tabc_cp.wait()
    tabs_cp.wait()
    for si in range(_NST):
        p = si % 2
        in_copy(si, p).wait()
        if si >= 2:
            out_copy(si - 2, p).wait()
        compute(p)
        out_copy(si, p).start()
        if si + 2 < _NST:
            in_copy(si + 2, p).start()
    out_copy(_NST - 2, (_NST - 2) % 2).wait()
    out_copy(_NST - 1, (_NST - 1) % 2).wait()


def kernel(z, constellation):
    zt = z.T
    ct = constellation.T
    out3 = pl.kernel(
        _psk_body,
        out_type=jax.ShapeDtypeStruct((_T, 2 * _B // 128, 128), jnp.float32),
        mesh=plsc.VectorSubcoreMesh(
            core_axis_name="c", subcore_axis_name="s",
            num_cores=_NC, num_subcores=_NS,
        ),
        scratch_types=(
            [pltpu.VMEM((16,), jnp.float32)] * 2
            + [pltpu.VMEM((_RT, _BW), jnp.int32)] * 2
            + [pltpu.VMEM((_RT, 8, 128), jnp.float32)] * 2
            + [pltpu.SemaphoreType.DMA] * 4
        ),
        compiler_params=pltpu.CompilerParams(
            needs_layout_passes=False, use_tc_tiling_on_sc=True,
        ),
    )(zt, ct)
    out = out3.reshape(_T, 128, 2, 128).transpose(1, 3, 0, 2).reshape(_B, _T, 2)
    return out


# speedup vs baseline: 1.0571x; 1.0036x over previous
import jax
import jax.numpy as jnp
from jax import lax
from jax.experimental import pallas as pl
from jax.experimental.pallas import tpu as pltpu
from jax.experimental.pallas import tpu_sc as plsc

_NC, _NS = 2, 16
_NW = _NC * _NS
_B, _T = 16384, 200
_BW = _B // _NW
_RT = 40
_NST = _T // _RT
_G = _RT * (_BW // 16)


def _psk_body(zt_hbm, ct_hbm, out_hbm, tabc_v, tabs_v,
              z0, z1, o0, o1, si0, si1, so0, so1, tsem):
    wid = lax.axis_index("s") * _NC + lax.axis_index("c")
    b0 = wid * _BW

    zbuf, obuf = (z0, z1), (o0, o1)
    zsem, osem = (si0, si1), (so0, so1)

    def in_copy(si, p):
        return pltpu.make_async_copy(
            zt_hbm.at[pl.ds(si * _RT, _RT), pl.ds(b0, _BW)], zbuf[p], zsem[p])

    def out_copy(si, p):
        return pltpu.make_async_copy(
            obuf[p], out_hbm.at[pl.ds(si * _RT, _RT), pl.ds(8 * wid, 8), :],
            osem[p])

    def compute(p):
        zv_ref, ov_ref = zbuf[p], obuf[p]

        @plsc.parallel_loop(0, _G, unroll=8)
        def _grp(i):
            t2 = i >> 5
            g = i & 31
            zv = zv_ref[t2, pl.ds(g * 16, 16)]
            cv = plsc.load_gather(tabc_v, [zv])
            sv = plsc.load_gather(tabs_v, [zv])
            bt = g >> 3
            j = g & 7
            ov_ref[t2, 2 * bt, pl.ds(j * 16, 16)] = cv
            ov_ref[t2, 2 * bt + 1, pl.ds(j * 16, 16)] = sv

    tabc_cp = pltpu.make_async_copy(ct_hbm.at[0], tabc_v, tsem)
    tabs_cp = pltpu.make_async_copy(ct_hbm.at[1], tabs_v, tsem)
    tabc_cp.start()
    tabs_cp.start()
    in_copy(0, 0).start()
    in_copy(1, 1).start()
    tabc_cp.wait()
    tabs_cp.wait()
    for si in range(_NST):
        p = si % 2
        in_copy(si, p).wait()
        if si >= 2:
            out_copy(si - 2, p).wait()
        compute(p)
        out_copy(si, p).start()
        if si + 2 < _NST:
            in_copy(si + 2, p).start()
    out_copy(_NST - 2, (_NST - 2) % 2).wait()
    out_copy(_NST - 1, (_NST - 1) % 2).wait()


def kernel(z, constellation):
    zt = z.T
    ct = constellation.T
    out3 = pl.kernel(
        _psk_body,
        out_type=jax.ShapeDtypeStruct((_T, 2 * _B // 128, 128), jnp.float32),
        mesh=plsc.VectorSubcoreMesh(
            core_axis_name="c", subcore_axis_name="s",
            num_cores=_NC, num_subcores=_NS,
        ),
        scratch_types=(
            [pltpu.VMEM((16,), jnp.float32)] * 2
            + [pltpu.VMEM((_RT, _BW), jnp.int32)] * 2
            + [pltpu.VMEM((_RT, 8, 128), jnp.float32)] * 2
            + [pltpu.SemaphoreType.DMA] * 5
        ),
        compiler_params=pltpu.CompilerParams(
            needs_layout_passes=False, use_tc_tiling_on_sc=True,
        ),
    )(zt, ct)
    out = out3.reshape(_T, 128, 2, 128).transpose(1, 3, 0, 2).reshape(_B, _T, 2)
    return out
